# baseline (speedup 1.0000x reference)
import functools

import jax
import jax.numpy as jnp
from jax import lax
from jax.experimental import pallas as pl
from jax.experimental.pallas import tpu as pltpu
from jax.experimental.pallas import tpu_sc as plsc

D_MODEL = 64
CHUNK = 800


def _make_sc_gather(n_rows: int):
    info = plsc.get_sparse_core_info()
    num_workers = info.num_cores * info.num_subcores
    per_w = n_rows // num_workers
    assert per_w * num_workers == n_rows
    chunk = CHUNK
    num_chunks = per_w // chunk
    assert num_chunks * chunk == per_w

    mesh = plsc.VectorSubcoreMesh(core_axis_name="c", subcore_axis_name="s")

    @functools.partial(
        pl.kernel,
        mesh=mesh,
        out_type=jax.ShapeDtypeStruct((n_rows, D_MODEL), jnp.float32),
        scratch_types=[
            pltpu.VMEM((chunk,), jnp.int32),
            pltpu.VMEM((chunk, D_MODEL), jnp.float32),
            pltpu.SemaphoreType.DMA,
        ],
        compiler_params=pltpu.CompilerParams(use_tc_tiling_on_sc=False),
    )
    def gather_kernel(idx_hbm, table_hbm, out_hbm, idx_v, rows_v, sem):
        wid = lax.axis_index("s") * info.num_cores + lax.axis_index("c")
        wbase = wid * per_w

        def body(g, carry):
            base = wbase + g * chunk
            pltpu.sync_copy(idx_hbm.at[pl.ds(base, chunk)], idx_v)
            pltpu.async_copy(table_hbm.at[idx_v], rows_v, sem).wait()
            pltpu.sync_copy(rows_v, out_hbm.at[pl.ds(base, chunk)])
            return carry

        lax.fori_loop(0, num_chunks, body, 0)

    return gather_kernel


def kernel(x, table):
    b, h = x.shape
    n = b * h
    idx = x.reshape(n).astype(jnp.int32)
    out = _make_sc_gather(n)(idx, table)
    return out.reshape(b, h, D_MODEL)


# speedup vs baseline: 1.0208x; 1.0208x over previous
import functools

import jax
import jax.numpy as jnp
from jax import lax
from jax.experimental import pallas as pl
from jax.experimental.pallas import tpu as pltpu
from jax.experimental.pallas import tpu_sc as plsc

D_MODEL = 64
NBUF = 4
CHUNK = 400


def _make_sc_gather(n_rows: int):
    info = plsc.get_sparse_core_info()
    num_workers = info.num_cores * info.num_subcores
    per_w = n_rows // num_workers
    assert per_w * num_workers == n_rows
    chunk = CHUNK
    nc = per_w // chunk
    assert nc * chunk == per_w and nc % NBUF == 0 and nc >= 2 * NBUF

    mesh = plsc.VectorSubcoreMesh(core_axis_name="c", subcore_axis_name="s")

    @functools.partial(
        pl.kernel,
        mesh=mesh,
        out_type=jax.ShapeDtypeStruct((n_rows, D_MODEL), jnp.float32),
        scratch_types=[
            pltpu.VMEM((per_w,), jnp.int32),
            pltpu.VMEM((NBUF, chunk, D_MODEL), jnp.float32),
        ]
        + [pltpu.SemaphoreType.DMA] * (2 * NBUF),
        compiler_params=pltpu.CompilerParams(use_tc_tiling_on_sc=False),
    )
    def gather_kernel(idx_hbm, table_hbm, out_hbm, idx_v, rows_v, *sems):
        gsem = sems[:NBUF]
        osem = sems[NBUF:]
        wid = lax.axis_index("s") * info.num_cores + lax.axis_index("c")
        wbase = wid * per_w

        pltpu.sync_copy(idx_hbm.at[pl.ds(wbase, per_w)], idx_v)

        def start_gather(g, b):
            pltpu.make_async_copy(
                table_hbm.at[idx_v.at[pl.ds(g * chunk, chunk)]],
                rows_v.at[b],
                gsem[b],
            ).start()

        def start_write(g, b):
            pltpu.make_async_copy(
                rows_v.at[b],
                out_hbm.at[pl.ds(wbase + g * chunk, chunk)],
                osem[b],
            ).start()

        def wait_chunk(sem):
            pltpu.make_async_copy(
                out_hbm.at[pl.ds(wbase, chunk)], rows_v.at[0], sem
            ).wait()

        start_gather(0, 0)
        start_gather(1, 1)
        wait_chunk(gsem[0])
        start_write(0, 0)
        start_gather(2, 2)
        wait_chunk(gsem[1])
        start_write(1, 1)
        start_gather(3, 3)

        def body(i, carry):
            for b in range(NBUF):
                g = NBUF * i + b
                wait_chunk(gsem[(b + 2) % NBUF])
                start_write(g - 2, (b + 2) % NBUF)
                wait_chunk(osem[b])
                start_gather(g, b)
            return carry

        lax.fori_loop(1, nc // NBUF, body, 0)

        wait_chunk(gsem[(nc - 2) % NBUF])
        start_write(nc - 2, (nc - 2) % NBUF)
        wait_chunk(gsem[(nc - 1) % NBUF])
        start_write(nc - 1, (nc - 1) % NBUF)
        for b in range(NBUF):
            wait_chunk(osem[b])

    return gather_kernel


def kernel(x, table):
    b, h = x.shape
    n = b * h
    idx = x.reshape(n).astype(jnp.int32)
    out = _make_sc_gather(n)(idx, table)
    return out.reshape(b, h, D_MODEL)
